# Initial kernel scaffold; baseline (speedup 1.0000x reference)
import jax
import jax.numpy as jnp
from jax.experimental import pallas as pl


def kernel(x, grid_points):
    raise NotImplementedError("write your pallas kernel here")


# speedup vs baseline: 5.3428x; 5.3428x over previous
import functools

import jax
import jax.numpy as jnp
from jax import lax
from jax.experimental import pallas as pl
from jax.experimental.pallas import tpu as pltpu
from jax.experimental.pallas import tpu_sc as plsc

B = 4
N = 2048
W = 32
NBINS = W ** 3
L = 16
NPT = N // L
NBG = NBINS // L

_mesh = plsc.VectorSubcoreMesh(core_axis_name="c", subcore_axis_name="s")


def _bf16_rne(x):
    u = lax.bitcast_convert_type(x, jnp.int32)
    lsb = jnp.bitwise_and(lax.shift_right_logical(u, 16), 1)
    r = jnp.bitwise_and(u + 0x7FFF + lsb, jnp.int32(-65536))
    return lax.bitcast_convert_type(r, jnp.float32)


def _cell(bx):
    t = (bx + 1.0) * 16.0
    q = t.astype(jnp.int32)
    q = jnp.minimum(jnp.maximum(q, 0), 31)
    lo = q.astype(jnp.float32) * 0.0625 - 1.0
    hi = (q + 1).astype(jnp.float32) * 0.0625 - 1.0
    q = (q - jnp.where(bx <= lo, 1, 0)) + jnp.where(bx > hi, 1, 0)
    q = jnp.minimum(jnp.maximum(q, 0), 31)
    hi2 = (q + 1).astype(jnp.float32) * 0.0625 - 1.0
    tie = jnp.logical_and(bx == hi2, q < 31)
    return q, tie


@functools.partial(
    pl.kernel,
    out_type=(
        jax.ShapeDtypeStruct((B, N), jnp.int32),
        jax.ShapeDtypeStruct((B, NBINS), jnp.float32),
    ),
    mesh=_mesh,
    compiler_params=pltpu.CompilerParams(needs_layout_passes=False),
    scratch_types=[
        pltpu.VMEM((3 * N,), jnp.float32),
        pltpu.VMEM((N,), jnp.int32),
        pltpu.VMEM((NBINS,), jnp.int32),
        pltpu.VMEM((NBINS,), jnp.float32),
        pltpu.VMEM((N,), jnp.int32),
        pltpu.VMEM((NBINS,), jnp.int32),
    ],
)
def _sc_quantize(x_hbm, idx_out, vox_out, coords_v, idx_v, counts_v, vox_v,
                 sorted_v, tmp_v):
    wid = lax.axis_index("s") * 2 + lax.axis_index("c")

    if True:
        b = wid & (B - 1)
        pltpu.sync_copy(x_hbm.at[b], coords_v)

        def memset_bins(i, carry):
            counts_v[pl.ds(i * L, L)] = jnp.zeros((L,), jnp.int32)
            return carry

        lax.fori_loop(0, NBG, memset_bins, 0)

        def memset_sorted(i, carry):
            sorted_v[pl.ds(i * L, L)] = jnp.zeros((L,), jnp.int32)
            return carry

        lax.fori_loop(0, NPT, memset_sorted, 0)

        def memset_vox(i, carry):
            vox_v[pl.ds(i * L, L)] = jnp.zeros((L,), jnp.float32)
            return carry

        lax.fori_loop(0, NBG, memset_vox, 0)

        def quant(i, carry):
            x0 = coords_v[pl.ds(i * L, L)]
            x1 = coords_v[pl.ds(N + i * L, L)]
            x2 = coords_v[pl.ds(2 * N + i * L, L)]
            b0 = _bf16_rne(x0)
            b1 = _bf16_rne(x1)
            b2 = _bf16_rne(x2)
            q0, t0 = _cell(b0)
            q1, t1 = _cell(b1)
            q2, t2 = _cell(b2)
            s1 = (x0 * x0 + x1 * x1) + x2 * x2
            best_d = None
            best_i = None
            for sel in range(8):
                qx = q0 + jnp.where(t0, 1, 0) if sel & 4 else q0
                qy = q1 + jnp.where(t1, 1, 0) if sel & 2 else q1
                qz = q2 + jnp.where(t2, 1, 0) if sel & 1 else q2
                cx = (qx.astype(jnp.float32) + 0.5) * 0.0625 - 1.0
                cy = (qy.astype(jnp.float32) + 0.5) * 0.0625 - 1.0
                cz = (qz.astype(jnp.float32) + 0.5) * 0.0625 - 1.0
                s2 = (cx * cx + cy * cy) + cz * cz
                m = (b0 * cx + b1 * cy) + b2 * cz
                d = (s1 + s2) - 2.0 * m
                ii = (qx * 1024 + qy * 32) + qz
                if best_d is None:
                    best_d, best_i = d, ii
                else:
                    better = jnp.logical_or(
                        d < best_d, jnp.logical_and(d == best_d, ii < best_i))
                    best_d = jnp.where(better, d, best_d)
                    best_i = jnp.where(better, ii, best_i)
            idx_v[pl.ds(i * L, L)] = best_i
            return carry

        lax.fori_loop(0, NPT, quant, 0)

        def hist(i, carry):
            iota = lax.iota(jnp.int32, L)
            k = idx_v[pl.ds(i * L, L)]
            ones = jnp.full((L,), 1, jnp.int32)

            def rnd(j, rem):
                m = rem > 0
                plsc.store_scatter(tmp_v, [k], iota, mask=m)
                back = plsc.load_gather(tmp_v, [k], mask=m)
                win = jnp.logical_and(m, back == iota)
                plsc.addupdate_scatter(counts_v, [k], ones, mask=win)
                return jnp.where(win, 0, rem)

            lax.fori_loop(0, L, rnd, ones)
            return carry

        lax.fori_loop(0, NPT, hist, 0)

        def binpass(i, carry):
            iota = lax.iota(jnp.int32, L)
            c16 = counts_v[pl.ds(i * L, L)]
            occ = c16 > 0
            vox_v[pl.ds(i * L, L)] = jnp.where(occ, jnp.float32(0.8),
                                               jnp.float32(-0.8))
            incl = plsc.cumsum(c16) + carry
            excl = incl - c16
            pos = jnp.minimum(jnp.maximum(excl, 0), N - 1)
            plsc.store_scatter(sorted_v, [pos], iota + i * L, mask=occ)
            return jnp.max(incl)

        lax.fori_loop(0, NBG, binpass, jnp.int32(0))

        def pmax(i, carry):
            v = sorted_v[pl.ds(i * L, L)]
            inc = jnp.maximum(plsc.cummax(v), carry)
            sorted_v[pl.ds(i * L, L)] = inc
            return jnp.max(inc)

        lax.fori_loop(0, NPT, pmax, jnp.int32(0))

        pltpu.sync_copy(sorted_v, idx_out.at[b])
        pltpu.sync_copy(vox_v, vox_out.at[b])


def kernel(x, grid_points):
    del grid_points
    xt = jnp.transpose(x, (0, 2, 1)).reshape(B, 3 * N)
    idx_sorted, vox = _sc_quantize(xt)
    voxel = vox.reshape(B, 1, W, W, W)
    return idx_sorted, voxel


# speedup vs baseline: 14.1628x; 2.6508x over previous
import functools

import jax
import jax.numpy as jnp
from jax import lax
from jax.experimental import pallas as pl
from jax.experimental.pallas import tpu as pltpu
from jax.experimental.pallas import tpu_sc as plsc

B = 4
N = 2048
W = 32
NBINS = W ** 3
L = 16
NSL = 8
SLB = NBINS // NSL
NPT = N // L
GSL = NPT // NSL
SGB = SLB // L

_mesh = plsc.VectorSubcoreMesh(core_axis_name="c", subcore_axis_name="s")


def _bf16_rne(x):
    u = lax.bitcast_convert_type(x, jnp.int32)
    lsb = jnp.bitwise_and(lax.shift_right_logical(u, 16), 1)
    r = jnp.bitwise_and(u + 0x7FFF + lsb, jnp.int32(-65536))
    return lax.bitcast_convert_type(r, jnp.float32)


def _cell(bx):
    t = (bx + 1.0) * 16.0
    q = t.astype(jnp.int32)
    q = jnp.minimum(jnp.maximum(q, 0), 31)
    lo = q.astype(jnp.float32) * 0.0625 - 1.0
    hi = (q + 1).astype(jnp.float32) * 0.0625 - 1.0
    q = (q - jnp.where(bx <= lo, 1, 0)) + jnp.where(bx > hi, 1, 0)
    q = jnp.minimum(jnp.maximum(q, 0), 31)
    hi2 = (q + 1).astype(jnp.float32) * 0.0625 - 1.0
    tie = jnp.logical_and(bx == hi2, q < 31)
    return q, tie


@functools.partial(
    pl.kernel,
    out_type=(
        jax.ShapeDtypeStruct((B, N), jnp.int32),
        jax.ShapeDtypeStruct((B, NBINS), jnp.float32),
    ),
    mesh=_mesh,
    compiler_params=pltpu.CompilerParams(needs_layout_passes=False),
    scratch_types=[
        pltpu.VMEM((3 * N,), jnp.float32),
        pltpu.VMEM((N,), jnp.int32),
        pltpu.VMEM((4, SLB), jnp.int32),
        pltpu.VMEM((SLB,), jnp.int32),
        pltpu.VMEM((SLB,), jnp.int32),
        pltpu.VMEM((SLB,), jnp.float32),
        pltpu.VMEM((N,), jnp.int32),
        pltpu.VMEM((L,), jnp.int32),
        pltpu.VMEM((NSL, L), jnp.int32),
        pltpu.VMEM((NSL, N), jnp.int32),
        pltpu.VMEM_SHARED((2, N), jnp.int32),
        pltpu.VMEM_SHARED((2, NSL, L), jnp.int32),
        pltpu.VMEM_SHARED((2, NSL, N), jnp.int32),
    ],
)
def _sc_quantize(x_hbm, idx_out, vox_out, coords_v, idx_v, hist4_v, cnt_v,
                 excl_v, vox_v, starts_v, totpub_v, tot_v, merge_v, idx_sh,
                 tot_sh, starts_sh):
    cid = lax.axis_index("c")
    sid = lax.axis_index("s")
    lb = sid // NSL
    sl = sid % NSL
    b = cid * 2 + lb
    lo = sl * SLB

    pltpu.sync_copy(x_hbm.at[b], coords_v)

    def quant(g, carry):
        i = sl * GSL + g
        x0 = coords_v[pl.ds(i * L, L)]
        x1 = coords_v[pl.ds(N + i * L, L)]
        x2 = coords_v[pl.ds(2 * N + i * L, L)]
        b0 = _bf16_rne(x0)
        b1 = _bf16_rne(x1)
        b2 = _bf16_rne(x2)
        q0, t0 = _cell(b0)
        q1, t1 = _cell(b1)
        q2, t2 = _cell(b2)
        s1 = (x0 * x0 + x1 * x1) + x2 * x2
        best_d = None
        best_i = None
        for sel in range(8):
            qx = q0 + jnp.where(t0, 1, 0) if sel & 4 else q0
            qy = q1 + jnp.where(t1, 1, 0) if sel & 2 else q1
            qz = q2 + jnp.where(t2, 1, 0) if sel & 1 else q2
            cx = (qx.astype(jnp.float32) + 0.5) * 0.0625 - 1.0
            cy = (qy.astype(jnp.float32) + 0.5) * 0.0625 - 1.0
            cz = (qz.astype(jnp.float32) + 0.5) * 0.0625 - 1.0
            s2 = (cx * cx + cy * cy) + cz * cz
            m = (b0 * cx + b1 * cy) + b2 * cz
            d = (s1 + s2) - 2.0 * m
            ii = (qx * 1024 + qy * 32) + qz
            if best_d is None:
                best_d, best_i = d, ii
            else:
                better = jnp.logical_or(
                    d < best_d, jnp.logical_and(d == best_d, ii < best_i))
                best_d = jnp.where(better, d, best_d)
                best_i = jnp.where(better, ii, best_i)
        idx_v[pl.ds(i * L, L)] = best_i
        return carry

    lax.fori_loop(0, GSL, quant, 0)

    pltpu.sync_copy(idx_v.at[pl.ds(sl * GSL * L, GSL * L)],
                    idx_sh.at[lb, pl.ds(sl * GSL * L, GSL * L)])
    plsc.subcore_barrier()
    pltpu.sync_copy(idx_sh.at[lb], idx_v)

    def memset_hist(i, carry):
        hist4_v[0, pl.ds(i * L, L)] = jnp.zeros((L,), jnp.int32)
        hist4_v[1, pl.ds(i * L, L)] = jnp.zeros((L,), jnp.int32)
        hist4_v[2, pl.ds(i * L, L)] = jnp.zeros((L,), jnp.int32)
        hist4_v[3, pl.ds(i * L, L)] = jnp.zeros((L,), jnp.int32)
        return carry

    lax.fori_loop(0, SGB, memset_hist, 0)

    def memset_sorted(i, carry):
        starts_v[pl.ds(i * L, L)] = jnp.zeros((L,), jnp.int32)
        return carry

    lax.fori_loop(0, NPT, memset_sorted, 0)

    def hist(i, carry):
        iota = lax.iota(jnp.int32, L)
        ones = jnp.full((L,), 1, jnp.int32)
        k = idx_v[pl.ds(i * L, L)]
        kl = jnp.minimum(jnp.maximum(k - lo, 0), SLB - 1)
        inr = jnp.logical_and(k >= lo, k < lo + SLB)
        for r in range(4):
            lane = jnp.logical_and(iota >= 4 * r, iota < 4 * r + 4)
            msk = jnp.logical_and(inr, lane)
            row = jnp.minimum(jnp.maximum(iota - 4 * r, 0), 3)
            plsc.addupdate_scatter(hist4_v, [row, kl], ones, mask=msk)
        return carry

    lax.fori_loop(0, NPT, hist, 0)

    def fold(i, carry):
        c16 = ((hist4_v[0, pl.ds(i * L, L)] + hist4_v[1, pl.ds(i * L, L)])
               + (hist4_v[2, pl.ds(i * L, L)] + hist4_v[3, pl.ds(i * L, L)]))
        cnt_v[pl.ds(i * L, L)] = c16
        vox_v[pl.ds(i * L, L)] = jnp.where(c16 > 0, jnp.float32(0.8),
                                           jnp.float32(-0.8))
        incl = plsc.cumsum(c16) + carry
        excl_v[pl.ds(i * L, L)] = incl - c16
        return jnp.max(incl)

    total = lax.fori_loop(0, SGB, fold, jnp.int32(0))

    pltpu.sync_copy(vox_v, vox_out.at[b, pl.ds(lo, SLB)])

    totpub_v[pl.ds(0, L)] = jnp.zeros((L,), jnp.int32) + total
    pltpu.sync_copy(totpub_v, tot_sh.at[lb, sl])
    plsc.subcore_barrier()
    pltpu.sync_copy(tot_sh.at[lb], tot_v)
    offs = jnp.zeros((L,), jnp.int32)
    for j in range(NSL):
        w = jnp.where(j < sl, 1, 0)
        offs = offs + tot_v[j, pl.ds(0, L)] * w

    def scat(i, carry):
        iota = lax.iota(jnp.int32, L)
        c16 = cnt_v[pl.ds(i * L, L)]
        occ = c16 > 0
        pos = excl_v[pl.ds(i * L, L)] + offs
        pos = jnp.minimum(jnp.maximum(pos, 0), N - 1)
        val = (lo + i * L) + iota
        plsc.store_scatter(starts_v, [pos], val, mask=occ)
        return carry

    lax.fori_loop(0, SGB, scat, 0)

    pltpu.sync_copy(starts_v, starts_sh.at[lb, sl])
    plsc.subcore_barrier()

    @pl.when(sl == 0)
    def _():
        pltpu.sync_copy(starts_sh.at[lb], merge_v)

        def pmax(i, carry):
            m = merge_v[0, pl.ds(i * L, L)]
            for j in range(1, NSL):
                m = jnp.maximum(m, merge_v[j, pl.ds(i * L, L)])
            inc = jnp.maximum(plsc.cummax(m), carry)
            starts_v[pl.ds(i * L, L)] = inc
            return jnp.max(inc)

        lax.fori_loop(0, NPT, pmax, jnp.int32(0))
        pltpu.sync_copy(starts_v, idx_out.at[b])


def kernel(x, grid_points):
    del grid_points
    xt = jnp.transpose(x, (0, 2, 1)).reshape(B, 3 * N)
    idx_sorted, vox = _sc_quantize(xt)
    voxel = vox.reshape(B, 1, W, W, W)
    return idx_sorted, voxel


# speedup vs baseline: 14.1960x; 1.0023x over previous
import functools

import jax
import jax.numpy as jnp
from jax import lax
from jax.experimental import pallas as pl
from jax.experimental.pallas import tpu as pltpu
from jax.experimental.pallas import tpu_sc as plsc

B = 4
N = 2048
W = 32
NBINS = W ** 3
L = 16
NSL = 8
SLB = NBINS // NSL
NPT = N // L
GSL = NPT // NSL
SGB = SLB // L

_mesh = plsc.VectorSubcoreMesh(core_axis_name="c", subcore_axis_name="s")


def _bf16_rne(x):
    u = lax.bitcast_convert_type(x, jnp.int32)
    lsb = jnp.bitwise_and(lax.shift_right_logical(u, 16), 1)
    r = jnp.bitwise_and(u + 0x7FFF + lsb, jnp.int32(-65536))
    return lax.bitcast_convert_type(r, jnp.float32)


def _cell(bx):
    t = (bx + 1.0) * 16.0
    q = t.astype(jnp.int32)
    q = jnp.minimum(jnp.maximum(q, 0), 31)
    lo = q.astype(jnp.float32) * 0.0625 - 1.0
    hi = (q + 1).astype(jnp.float32) * 0.0625 - 1.0
    q = (q - jnp.where(bx <= lo, 1, 0)) + jnp.where(bx > hi, 1, 0)
    q = jnp.minimum(jnp.maximum(q, 0), 31)
    hi2 = (q + 1).astype(jnp.float32) * 0.0625 - 1.0
    tie = jnp.logical_and(bx == hi2, q < 31)
    return q, tie


@functools.partial(
    pl.kernel,
    out_type=(
        jax.ShapeDtypeStruct((B, N), jnp.int32),
        jax.ShapeDtypeStruct((B, NBINS), jnp.float32),
    ),
    mesh=_mesh,
    compiler_params=pltpu.CompilerParams(needs_layout_passes=False),
    scratch_types=[
        pltpu.VMEM((3 * N,), jnp.float32),
        pltpu.VMEM((N,), jnp.int32),
        pltpu.VMEM((4, SLB), jnp.int32),
        pltpu.VMEM((SLB,), jnp.int32),
        pltpu.VMEM((SLB,), jnp.int32),
        pltpu.VMEM((SLB,), jnp.float32),
        pltpu.VMEM((N,), jnp.int32),
        pltpu.VMEM((4, SGB), jnp.int32),
        pltpu.VMEM((SLB,), jnp.int32),
        pltpu.VMEM((L,), jnp.int32),
        pltpu.VMEM((NSL, L), jnp.int32),
        pltpu.VMEM((NSL, N), jnp.int32),
        pltpu.VMEM_SHARED((2, N), jnp.int32),
        pltpu.VMEM_SHARED((2, NSL, L), jnp.int32),
        pltpu.VMEM_SHARED((2, NSL, N), jnp.int32),
    ],
)
def _sc_quantize(x_hbm, idx_out, vox_out, coords_v, idx_v, hist4_v, cnt_v,
                 excl_v, vox_v, starts_v, hist4g_v, gexp_v, totpub_v, tot_v,
                 merge_v, idx_sh, tot_sh, starts_sh):
    cid = lax.axis_index("c")
    sid = lax.axis_index("s")
    lb = sid // NSL
    sl = sid % NSL
    b = cid * 2 + lb
    lo = sl * SLB

    pltpu.sync_copy(x_hbm.at[b], coords_v)

    def quant(g, carry):
        i = sl * GSL + g
        x0 = coords_v[pl.ds(i * L, L)]
        x1 = coords_v[pl.ds(N + i * L, L)]
        x2 = coords_v[pl.ds(2 * N + i * L, L)]
        b0 = _bf16_rne(x0)
        b1 = _bf16_rne(x1)
        b2 = _bf16_rne(x2)
        q0, t0 = _cell(b0)
        q1, t1 = _cell(b1)
        q2, t2 = _cell(b2)
        s1 = (x0 * x0 + x1 * x1) + x2 * x2
        best_d = None
        best_i = None
        for sel in range(8):
            qx = q0 + jnp.where(t0, 1, 0) if sel & 4 else q0
            qy = q1 + jnp.where(t1, 1, 0) if sel & 2 else q1
            qz = q2 + jnp.where(t2, 1, 0) if sel & 1 else q2
            cx = (qx.astype(jnp.float32) + 0.5) * 0.0625 - 1.0
            cy = (qy.astype(jnp.float32) + 0.5) * 0.0625 - 1.0
            cz = (qz.astype(jnp.float32) + 0.5) * 0.0625 - 1.0
            s2 = (cx * cx + cy * cy) + cz * cz
            m = (b0 * cx + b1 * cy) + b2 * cz
            d = (s1 + s2) - 2.0 * m
            ii = (qx * 1024 + qy * 32) + qz
            if best_d is None:
                best_d, best_i = d, ii
            else:
                better = jnp.logical_or(
                    d < best_d, jnp.logical_and(d == best_d, ii < best_i))
                best_d = jnp.where(better, d, best_d)
                best_i = jnp.where(better, ii, best_i)
        idx_v[pl.ds(i * L, L)] = best_i
        return carry

    lax.fori_loop(0, GSL, quant, 0)

    pltpu.sync_copy(idx_v.at[pl.ds(sl * GSL * L, GSL * L)],
                    idx_sh.at[lb, pl.ds(sl * GSL * L, GSL * L)])
    plsc.subcore_barrier()
    pltpu.sync_copy(idx_sh.at[lb], idx_v)

    def memset_hist(i, carry):
        hist4_v[0, pl.ds(i * L, L)] = jnp.zeros((L,), jnp.int32)
        hist4_v[1, pl.ds(i * L, L)] = jnp.zeros((L,), jnp.int32)
        hist4_v[2, pl.ds(i * L, L)] = jnp.zeros((L,), jnp.int32)
        hist4_v[3, pl.ds(i * L, L)] = jnp.zeros((L,), jnp.int32)
        return carry

    lax.fori_loop(0, SGB, memset_hist, 0)

    def memset_histg(i, carry):
        hist4g_v[0, pl.ds(i * L, L)] = jnp.zeros((L,), jnp.int32)
        hist4g_v[1, pl.ds(i * L, L)] = jnp.zeros((L,), jnp.int32)
        hist4g_v[2, pl.ds(i * L, L)] = jnp.zeros((L,), jnp.int32)
        hist4g_v[3, pl.ds(i * L, L)] = jnp.zeros((L,), jnp.int32)
        return carry

    lax.fori_loop(0, SGB // L, memset_histg, 0)

    def memset_sorted(i, carry):
        starts_v[pl.ds(i * L, L)] = jnp.zeros((L,), jnp.int32)
        return carry

    lax.fori_loop(0, NPT, memset_sorted, 0)

    def hist(i, carry):
        iota = lax.iota(jnp.int32, L)
        ones = jnp.full((L,), 1, jnp.int32)
        k = idx_v[pl.ds(i * L, L)]
        kl = jnp.minimum(jnp.maximum(k - lo, 0), SLB - 1)
        inr = jnp.logical_and(k >= lo, k < lo + SLB)
        kg = lax.shift_right_logical(kl, 4)
        for r in range(4):
            lane = jnp.logical_and(iota >= 4 * r, iota < 4 * r + 4)
            msk = jnp.logical_and(inr, lane)
            row = jnp.minimum(jnp.maximum(iota - 4 * r, 0), 3)
            plsc.addupdate_scatter(hist4_v, [row, kl], ones, mask=msk)
            plsc.addupdate_scatter(hist4g_v, [row, kg], ones, mask=msk)
        return carry

    lax.fori_loop(0, NPT, hist, 0)

    def gprefix(j, carry):
        iota = lax.iota(jnp.int32, L)
        g16 = ((hist4g_v[0, pl.ds(j * L, L)] + hist4g_v[1, pl.ds(j * L, L)])
               + (hist4g_v[2, pl.ds(j * L, L)] + hist4g_v[3, pl.ds(j * L, L)]))
        incl = plsc.cumsum(g16) + carry
        gex = incl - g16
        base = (j * L + iota) * L
        for c in range(L):
            plsc.store_scatter(gexp_v, [base + c], gex)
        return jnp.max(incl)

    total = lax.fori_loop(0, SGB // L, gprefix, jnp.int32(0))

    def fold(i, carry):
        c16 = ((hist4_v[0, pl.ds(i * L, L)] + hist4_v[1, pl.ds(i * L, L)])
               + (hist4_v[2, pl.ds(i * L, L)] + hist4_v[3, pl.ds(i * L, L)]))
        cnt_v[pl.ds(i * L, L)] = c16
        vox_v[pl.ds(i * L, L)] = jnp.where(c16 > 0, jnp.float32(0.8),
                                           jnp.float32(-0.8))
        incl = plsc.cumsum(c16)
        excl_v[pl.ds(i * L, L)] = incl - c16
        return carry

    lax.fori_loop(0, SGB, fold, jnp.int32(0))

    pltpu.sync_copy(vox_v, vox_out.at[b, pl.ds(lo, SLB)])

    totpub_v[pl.ds(0, L)] = jnp.zeros((L,), jnp.int32) + total
    pltpu.sync_copy(totpub_v, tot_sh.at[lb, sl])
    plsc.subcore_barrier()
    pltpu.sync_copy(tot_sh.at[lb], tot_v)
    offs = jnp.zeros((L,), jnp.int32)
    for j in range(NSL):
        w = jnp.where(j < sl, 1, 0)
        offs = offs + tot_v[j, pl.ds(0, L)] * w

    def scat(i, carry):
        iota = lax.iota(jnp.int32, L)
        c16 = cnt_v[pl.ds(i * L, L)]
        occ = c16 > 0
        pos = (excl_v[pl.ds(i * L, L)] + gexp_v[pl.ds(i * L, L)]) + offs
        pos = jnp.minimum(jnp.maximum(pos, 0), N - 1)
        val = (lo + i * L) + iota
        plsc.store_scatter(starts_v, [pos], val, mask=occ)
        return carry

    lax.fori_loop(0, SGB, scat, 0)

    pltpu.sync_copy(starts_v, starts_sh.at[lb, sl])
    plsc.subcore_barrier()

    @pl.when(sl == 0)
    def _():
        pltpu.sync_copy(starts_sh.at[lb], merge_v)

        def pmax(i, carry):
            m = merge_v[0, pl.ds(i * L, L)]
            for j in range(1, NSL):
                m = jnp.maximum(m, merge_v[j, pl.ds(i * L, L)])
            inc = jnp.maximum(plsc.cummax(m), carry)
            starts_v[pl.ds(i * L, L)] = inc
            return jnp.max(inc)

        lax.fori_loop(0, NPT, pmax, jnp.int32(0))
        pltpu.sync_copy(starts_v, idx_out.at[b])


def kernel(x, grid_points):
    del grid_points
    xt = jnp.transpose(x, (0, 2, 1)).reshape(B, 3 * N)
    idx_sorted, vox = _sc_quantize(xt)
    voxel = vox.reshape(B, 1, W, W, W)
    return idx_sorted, voxel
